# Initial kernel scaffold; baseline (speedup 1.0000x reference)
import jax
import jax.numpy as jnp
from jax.experimental import pallas as pl


def kernel(logits):
    raise NotImplementedError("write your pallas kernel here")


# speedup vs baseline: 7.2172x; 7.2172x over previous
import math

import jax
import jax.numpy as jnp
import numpy as np
from jax import lax
from jax.experimental import pallas as pl
from jax.experimental.pallas import tpu as pltpu
from jax.experimental.pallas import tpu_sc as plsc

B = 32
N = 100000
K = int(math.ceil((1.0 - 0.9) * N))
L = 16
NV = N // L
CHUNK = 10000
NCV = CHUNK // L
NCH = N // CHUNK
NBINS = 256
HIST = NBINS * L

_SIGN = np.uint32(0x80000000)
_NEG_INF = np.float32(-np.inf)


def _monotone_u32(x):
    ub = plsc.bitcast(x, jnp.uint32)
    return jnp.where(x < 0.0, ~ub, ub | _SIGN)


def _body(logits_hbm, gumbel_hbm, probs_hbm, samp_hbm,
          row_v, hist_v, gbuf_v, obuf_v, samp_v):
    info = plsc.get_sparse_core_info()
    nc = info.num_cores
    wid = lax.axis_index("s") * nc + lax.axis_index("c")
    row = wid

    zero = wid * 0
    zvec = jnp.broadcast_to(zero, (L,))
    zvecf = zvec.astype(jnp.float32)
    ones = zvec + 1
    lane = (plsc.cumsum(zvecf + 1.0) - 1.0).astype(jnp.int32)
    ninf = zvecf + _NEG_INF

    pltpu.sync_copy(logits_hbm.at[pl.ds(row * N, N)], row_v)

    def zero_hist(i, _):
        hist_v[pl.ds(i * L, L)] = zvec
        return 0

    def hist_pass(shift, prefix):
        def body(i, mx):
            x = row_v[pl.ds(i * L, L)]
            u = _monotone_u32(x)
            dig = ((u >> shift) & np.uint32(0xFF)).astype(jnp.int32)
            idx = dig * L + lane
            if shift == 24:
                plsc.addupdate_scatter(hist_v, [idx], ones)
                mx = jnp.maximum(mx, x)
            else:
                ok = (u >> (shift + 8)) == prefix
                plsc.addupdate_scatter(hist_v, [idx], ones, mask=ok)
            return mx
        return body

    def scan_digits(r):
        def body(i, carry):
            csum, dig, r = carry
            j = 255 - i
            h = hist_v[pl.ds(j * L, L)]
            cj = jnp.sum(h.astype(jnp.float32)).astype(jnp.int32)
            hit = (dig < 0) & (csum + cj >= r)
            new_dig = jnp.where(hit, j, dig)
            new_r = jnp.where(hit, r - csum, r)
            new_csum = jnp.where((dig < 0) & (~hit), csum + cj, csum)
            return new_csum, new_dig, new_r
        csum, dig, r = lax.fori_loop(
            0, NBINS, body, (np.int32(0), np.int32(-1), r))
        return dig, r

    prefix = np.uint32(0)
    r = np.int32(K)
    macc = ninf
    for p in range(4):
        shift = 24 - 8 * p
        lax.fori_loop(0, NBINS, zero_hist, 0)
        if p == 0:
            macc = lax.fori_loop(0, NV, hist_pass(shift, prefix), macc)
        else:
            lax.fori_loop(0, NV, lambda i, c, f=hist_pass(shift, prefix):
                          (f(i, None), c)[1], 0)
        dig, r = scan_digits(r)
        prefix = (prefix << 8) | dig.astype(jnp.uint32)

    tb = jnp.where((prefix >> 31) == np.uint32(1),
                   prefix ^ _SIGN, ~prefix)
    tvec = plsc.bitcast(jnp.broadcast_to(tb, (L,)), jnp.float32)
    mx = jnp.max(macc)

    def sum_body(i, acc):
        x = row_v[pl.ds(i * L, L)]
        e = jnp.exp(x - mx)
        return acc + jnp.where(x >= tvec, e, 0.0)
    sacc = lax.fori_loop(0, NV, sum_body, zvecf)
    s = jnp.sum(sacc)

    bval = ninf
    bidx = zvec
    for c in range(NCH):
        pltpu.sync_copy(gumbel_hbm.at[pl.ds(row * N + c * CHUNK, CHUNK)], gbuf_v)

        def chunk_body(i, carry, c=c):
            bval, bidx = carry
            x = row_v[pl.ds(c * CHUNK + i * L, L)]
            g = gbuf_v[pl.ds(i * L, L)]
            kept = x >= tvec
            pr = jnp.where(kept, jnp.exp(x - mx) / s, 0.0)
            obuf_v[pl.ds(i * L, L)] = pr
            score = jnp.where(kept, x + g, _NEG_INF)
            better = score > bval
            bval = jnp.where(better, score, bval)
            bidx = jnp.where(better, c * CHUNK + i * L + lane, bidx)
            return bval, bidx

        bval, bidx = lax.fori_loop(0, NCV, chunk_body, (bval, bidx))
        pltpu.sync_copy(obuf_v, probs_hbm.at[pl.ds(row * N + c * CHUNK, CHUNK)])

    mbest = jnp.max(bval)
    cand = jnp.where(bval == mbest, bidx.astype(jnp.float32), np.float32(3e7))
    samp = jnp.min(cand).astype(jnp.int32)
    samp_v[...] = jnp.broadcast_to(samp, (L,))
    pltpu.sync_copy(samp_v, samp_hbm.at[pl.ds(row * L, L)])


_mesh = plsc.VectorSubcoreMesh(core_axis_name="c", subcore_axis_name="s")

_sc_call = pl.kernel(
    _body,
    out_type=(
        jax.ShapeDtypeStruct((B * N,), jnp.float32),
        jax.ShapeDtypeStruct((B * L,), jnp.int32),
    ),
    mesh=_mesh,
    compiler_params=pltpu.CompilerParams(needs_layout_passes=False),
    scratch_types=[
        pltpu.VMEM((N,), jnp.float32),
        pltpu.VMEM((HIST,), jnp.int32),
        pltpu.VMEM((CHUNK,), jnp.float32),
        pltpu.VMEM((CHUNK,), jnp.float32),
        pltpu.VMEM((L,), jnp.int32),
    ],
)


@jax.jit
def kernel(logits):
    gumbel = jax.random.gumbel(jax.random.key(1), (B, N), jnp.float32)
    probs, samp = _sc_call(logits.reshape(-1), gumbel.reshape(-1))
    return probs.reshape(B, N), samp.reshape(B, L)[:, :1]


# speedup vs baseline: 8.1621x; 1.1309x over previous
import math

import jax
import jax.numpy as jnp
import numpy as np
from jax import lax
from jax.experimental import pallas as pl
from jax.experimental.pallas import tpu as pltpu
from jax.experimental.pallas import tpu_sc as plsc

B = 32
N = 100000
K = int(math.ceil((1.0 - 0.9) * N))
L = 16
NV = N // L
CHUNK = 10000
NCV = CHUNK // L
NCH = N // CHUNK
NBINS = 256
HIST = NBINS * L
UNROLL = 5

_SIGN = np.uint32(0x80000000)
_NEG_INF = np.float32(-np.inf)


def _monotone_u32(x):
    ub = plsc.bitcast(x, jnp.uint32)
    return jnp.where(x < 0.0, ~ub, ub | _SIGN)


def _body(logits_hbm, gumbel_hbm, probs_hbm, samp_hbm,
          row_v, hist_v, gbuf_v, obuf_v, samp_v):
    info = plsc.get_sparse_core_info()
    nc = info.num_cores
    wid = lax.axis_index("s") * nc + lax.axis_index("c")
    row = wid

    zero = wid * 0
    zvec = jnp.broadcast_to(zero, (L,))
    zvecf = zvec.astype(jnp.float32)
    ones = zvec + 1
    lane = (plsc.cumsum(zvecf + 1.0) - 1.0).astype(jnp.int32)
    ninf = zvecf + _NEG_INF

    pltpu.sync_copy(logits_hbm.at[pl.ds(row * N, N)], row_v)

    def zero_hist(i, _):
        hist_v[pl.ds(i * L, L)] = zvec
        return 0

    def hist_pass(shift, prefix):
        def body(i, mx):
            for j in range(UNROLL):
                x = row_v[pl.ds((i * UNROLL + j) * L, L)]
                u = _monotone_u32(x)
                dig = ((u >> shift) & np.uint32(0xFF)).astype(jnp.int32)
                idx = dig * L + lane
                if shift == 24:
                    plsc.addupdate_scatter(hist_v, [idx], ones)
                    mx = jnp.maximum(mx, x)
                else:
                    ok = (u >> (shift + 8)) == prefix
                    plsc.addupdate_scatter(hist_v, [idx], ones, mask=ok)
            return mx
        return body

    def scan_digits(r):
        def body(i, carry):
            csum, dig, r = carry
            j = 255 - i
            h = hist_v[pl.ds(j * L, L)]
            cj = jnp.sum(h.astype(jnp.float32)).astype(jnp.int32)
            hit = (dig < 0) & (csum + cj >= r)
            new_dig = jnp.where(hit, j, dig)
            new_r = jnp.where(hit, r - csum, r)
            new_csum = jnp.where((dig < 0) & (~hit), csum + cj, csum)
            return new_csum, new_dig, new_r
        csum, dig, r = lax.fori_loop(
            0, NBINS, body, (np.int32(0), np.int32(-1), r))
        return dig, r

    prefix = np.uint32(0)
    r = np.int32(K)
    macc = ninf
    for p in range(4):
        shift = 24 - 8 * p
        lax.fori_loop(0, NBINS, zero_hist, 0)
        if p == 0:
            macc = lax.fori_loop(0, NV // UNROLL, hist_pass(shift, prefix), macc)
        else:
            lax.fori_loop(0, NV // UNROLL, lambda i, c, f=hist_pass(shift, prefix):
                          (f(i, None), c)[1], 0)
        dig, r = scan_digits(r)
        prefix = (prefix << 8) | dig.astype(jnp.uint32)

    tb = jnp.where((prefix >> 31) == np.uint32(1),
                   prefix ^ _SIGN, ~prefix)
    tvec = plsc.bitcast(jnp.broadcast_to(tb, (L,)), jnp.float32)
    mx = jnp.max(macc)

    def sum_body(i, accs):
        out = []
        for j in range(UNROLL):
            x = row_v[pl.ds((i * UNROLL + j) * L, L)]
            e = jnp.exp(x - mx)
            out.append(accs[j] + jnp.where(x >= tvec, e, 0.0))
        return tuple(out)
    saccs = lax.fori_loop(0, NV // UNROLL, sum_body, (zvecf,) * UNROLL)
    sacc = saccs[0]
    for j in range(1, UNROLL):
        sacc = sacc + saccs[j]
    s = jnp.sum(sacc)

    bvals = (ninf,) * UNROLL
    bidxs = (zvec,) * UNROLL
    for c in range(NCH):
        pltpu.sync_copy(gumbel_hbm.at[pl.ds(row * N + c * CHUNK, CHUNK)], gbuf_v)

        def chunk_body(i, carry, c=c):
            bvals, bidxs = carry
            bvals, bidxs = list(bvals), list(bidxs)
            for j in range(UNROLL):
                off = (i * UNROLL + j) * L
                x = row_v[pl.ds(c * CHUNK + off, L)]
                g = gbuf_v[pl.ds(off, L)]
                kept = x >= tvec
                pr = jnp.where(kept, jnp.exp(x - mx) / s, 0.0)
                obuf_v[pl.ds(off, L)] = pr
                score = jnp.where(kept, x + g, _NEG_INF)
                better = score > bvals[j]
                bvals[j] = jnp.where(better, score, bvals[j])
                bidxs[j] = jnp.where(better, c * CHUNK + off + lane, bidxs[j])
            return tuple(bvals), tuple(bidxs)

        bvals, bidxs = lax.fori_loop(0, NCV // UNROLL, chunk_body,
                                     (bvals, bidxs))
        pltpu.sync_copy(obuf_v, probs_hbm.at[pl.ds(row * N + c * CHUNK, CHUNK)])

    bval, bidx = bvals[0], bidxs[0]
    for j in range(1, UNROLL):
        better = bvals[j] > bval
        bval = jnp.where(better, bvals[j], bval)
        bidx = jnp.where(better, bidxs[j], bidx)
    mbest = jnp.max(bval)
    cand = jnp.where(bval == mbest, bidx.astype(jnp.float32), np.float32(3e7))
    samp = jnp.min(cand).astype(jnp.int32)
    samp_v[...] = jnp.broadcast_to(samp, (L,))
    pltpu.sync_copy(samp_v, samp_hbm.at[pl.ds(row * L, L)])


_mesh = plsc.VectorSubcoreMesh(core_axis_name="c", subcore_axis_name="s")

_sc_call = pl.kernel(
    _body,
    out_type=(
        jax.ShapeDtypeStruct((B * N,), jnp.float32),
        jax.ShapeDtypeStruct((B * L,), jnp.int32),
    ),
    mesh=_mesh,
    compiler_params=pltpu.CompilerParams(needs_layout_passes=False),
    scratch_types=[
        pltpu.VMEM((N,), jnp.float32),
        pltpu.VMEM((HIST,), jnp.int32),
        pltpu.VMEM((CHUNK,), jnp.float32),
        pltpu.VMEM((CHUNK,), jnp.float32),
        pltpu.VMEM((L,), jnp.int32),
    ],
)


@jax.jit
def kernel(logits):
    gumbel = jax.random.gumbel(jax.random.key(1), (B, N), jnp.float32)
    probs, samp = _sc_call(logits.reshape(-1), gumbel.reshape(-1))
    return probs.reshape(B, N), samp.reshape(B, L)[:, :1]


# speedup vs baseline: 14.9959x; 1.8373x over previous
import math

import jax
import jax.numpy as jnp
import numpy as np
from jax import lax
from jax.experimental import pallas as pl
from jax.experimental.pallas import tpu as pltpu
from jax.experimental.pallas import tpu_sc as plsc

B = 32
N = 100000
K = int(math.ceil((1.0 - 0.9) * N))
L = 16
NV = N // L
CHUNK = 10000
NCV = CHUNK // L
NCH = N // CHUNK
NBINS = 256
HIST = NBINS * L
UNROLL = 10
FUNROLL = 5

_SIGN = np.uint32(0x80000000)
_NEG_INF = np.float32(-np.inf)


def _monotone_u32(x):
    ub = plsc.bitcast(x, jnp.uint32)
    return jnp.where(x < 0.0, ~ub, ub | _SIGN)


def _body(logits_hbm, gumbel_hbm, probs_hbm, samp_hbm,
          row_v, hist_v, gbuf_v, obuf_v, samp_v):
    info = plsc.get_sparse_core_info()
    nc = info.num_cores
    wid = lax.axis_index("s") * nc + lax.axis_index("c")
    row = wid

    zero = wid * 0
    zvec = jnp.broadcast_to(zero, (L,))
    zvecf = zvec.astype(jnp.float32)
    ones = zvec + 1
    lane = (plsc.cumsum(zvecf + 1.0) - 1.0).astype(jnp.int32)
    ninf = zvecf + _NEG_INF

    pltpu.sync_copy(logits_hbm.at[pl.ds(row * N, N)], row_v)

    def zero_hist(i, _):
        hist_v[pl.ds(i * L, L)] = zvec
        return 0

    def hist_pass(shift, prefix):
        def body(i, mx):
            x = row_v[pl.ds(i * L, L)]
            u = _monotone_u32(x)
            dig = ((u >> shift) & np.uint32(0xFF)).astype(jnp.int32)
            idx = dig * L + lane
            if shift == 24:
                plsc.addupdate_scatter(hist_v, [idx], ones)
                return jnp.maximum(mx, x)
            ok = (u >> (shift + 8)) == prefix
            plsc.addupdate_scatter(hist_v, [idx], ones, mask=ok)
            return mx
        return body

    def scan_digits(r):
        def body(i, carry):
            csum, dig, r = carry
            j = 255 - i
            h = hist_v[pl.ds(j * L, L)]
            cj = jnp.sum(h.astype(jnp.float32)).astype(jnp.int32)
            hit = (dig < 0) & (csum + cj >= r)
            new_dig = jnp.where(hit, j, dig)
            new_r = jnp.where(hit, r - csum, r)
            new_csum = jnp.where((dig < 0) & (~hit), csum + cj, csum)
            return new_csum, new_dig, new_r
        csum, dig, r = lax.fori_loop(
            0, NBINS, body, (np.int32(0), np.int32(-1), r))
        return dig, r

    prefix = np.uint32(0)
    r = np.int32(K)
    macc = ninf
    for p in range(4):
        shift = 24 - 8 * p
        lax.fori_loop(0, NBINS, zero_hist, 0)
        if p == 0:
            macc = plsc.parallel_loop(0, NV, 1, unroll=UNROLL, carry=macc)(
                hist_pass(shift, prefix))
        else:
            plsc.parallel_loop(0, NV, 1, unroll=UNROLL, carry=ninf)(
                hist_pass(shift, prefix))
        dig, r = scan_digits(r)
        prefix = (prefix << 8) | dig.astype(jnp.uint32)

    tb = jnp.where((prefix >> 31) == np.uint32(1),
                   prefix ^ _SIGN, ~prefix)
    tvec = plsc.bitcast(jnp.broadcast_to(tb, (L,)), jnp.float32)
    mx = jnp.max(macc)

    def sum_body(i, acc):
        x = row_v[pl.ds(i * L, L)]
        e = jnp.exp(x - mx)
        return acc + jnp.where(x >= tvec, e, 0.0)
    sacc = plsc.parallel_loop(0, NV, 1, unroll=UNROLL, carry=zvecf)(sum_body)
    s = jnp.sum(sacc)

    bvals, bidxs = ninf, zvec
    for c in range(NCH):
        pltpu.sync_copy(gumbel_hbm.at[pl.ds(row * N + c * CHUNK, CHUNK)], gbuf_v)

        def chunk_body(i, carry, c=c):
            bval, bidx = carry
            off = i * L
            x = row_v[pl.ds(c * CHUNK + off, L)]
            g = gbuf_v[pl.ds(off, L)]
            kept = x >= tvec
            pr = jnp.where(kept, jnp.exp(x - mx) / s, 0.0)
            obuf_v[pl.ds(off, L)] = pr
            score = jnp.where(kept, x + g, _NEG_INF)
            better = score > bval
            bval = jnp.where(better, score, bval)
            bidx = jnp.where(better, c * CHUNK + off + lane, bidx)
            return bval, bidx

        bvals, bidxs = plsc.parallel_loop(
            0, NCV, 1, unroll=FUNROLL, carry=(bvals, bidxs))(chunk_body)
        pltpu.sync_copy(obuf_v, probs_hbm.at[pl.ds(row * N + c * CHUNK, CHUNK)])

    bval, bidx = bvals, bidxs
    mbest = jnp.max(bval)
    cand = jnp.where(bval == mbest, bidx.astype(jnp.float32), np.float32(3e7))
    samp = jnp.min(cand).astype(jnp.int32)
    samp_v[...] = jnp.broadcast_to(samp, (L,))
    pltpu.sync_copy(samp_v, samp_hbm.at[pl.ds(row * L, L)])


_mesh = plsc.VectorSubcoreMesh(core_axis_name="c", subcore_axis_name="s")

_sc_call = pl.kernel(
    _body,
    out_type=(
        jax.ShapeDtypeStruct((B * N,), jnp.float32),
        jax.ShapeDtypeStruct((B * L,), jnp.int32),
    ),
    mesh=_mesh,
    compiler_params=pltpu.CompilerParams(needs_layout_passes=False),
    scratch_types=[
        pltpu.VMEM((N,), jnp.float32),
        pltpu.VMEM((HIST,), jnp.int32),
        pltpu.VMEM((CHUNK,), jnp.float32),
        pltpu.VMEM((CHUNK,), jnp.float32),
        pltpu.VMEM((L,), jnp.int32),
    ],
)


@jax.jit
def kernel(logits):
    gumbel = jax.random.gumbel(jax.random.key(1), (B, N), jnp.float32)
    probs, samp = _sc_call(logits.reshape(-1), gumbel.reshape(-1))
    return probs.reshape(B, N), samp.reshape(B, L)[:, :1]


# speedup vs baseline: 25.3085x; 1.6877x over previous
import math

import jax
import jax.numpy as jnp
import numpy as np
from jax import lax
from jax.experimental import pallas as pl
from jax.experimental.pallas import tpu as pltpu
from jax.experimental.pallas import tpu_sc as plsc

B = 32
N = 100000
K = int(math.ceil((1.0 - 0.9) * N))
L = 16
NV = N // L
CHUNK = 10000
NCV = CHUNK // L
NCH = N // CHUNK
NBINS = 256
HIST = NBINS * L
UNROLL = 10
FUNROLL = 5

_SIGN = np.uint32(0x80000000)
_NEG_INF = np.float32(-np.inf)

_mesh = plsc.VectorSubcoreMesh(core_axis_name="c", subcore_axis_name="s")
_params = pltpu.CompilerParams(needs_layout_passes=False)


def _monotone_u32(x):
    ub = plsc.bitcast(x, jnp.uint32)
    return jnp.where(x < 0.0, ~ub, ub | _SIGN)


def _body_a(logits_hbm, thr_hbm, row_v, hist_v, st_v):
    info = plsc.get_sparse_core_info()
    nc = info.num_cores
    wid = lax.axis_index("s") * nc + lax.axis_index("c")
    row = wid

    zero = wid * 0
    zvec = jnp.broadcast_to(zero, (L,))
    zvecf = zvec.astype(jnp.float32)
    ones = zvec + 1
    lane = (plsc.cumsum(zvecf + 1.0) - 1.0).astype(jnp.int32)
    ninf = zvecf + _NEG_INF

    pltpu.sync_copy(logits_hbm.at[pl.ds(row * N, N)], row_v)

    def zero_hist(i, _):
        hist_v[pl.ds(i * L, L)] = zvec
        return 0

    def hist_pass(shift, prefix):
        def body(i, mx):
            x = row_v[pl.ds(i * L, L)]
            u = _monotone_u32(x)
            dig = ((u >> shift) & np.uint32(0xFF)).astype(jnp.int32)
            idx = dig * L + lane
            if shift == 24:
                plsc.addupdate_scatter(hist_v, [idx], ones)
                return jnp.maximum(mx, x)
            ok = (u >> (shift + 8)) == prefix
            plsc.addupdate_scatter(hist_v, [idx], ones, mask=ok)
            return mx
        return body

    def scan_digits(r):
        def body(i, carry):
            csum, dig, r = carry
            j = 255 - i
            h = hist_v[pl.ds(j * L, L)]
            cj = jnp.sum(h.astype(jnp.float32)).astype(jnp.int32)
            hit = (dig < 0) & (csum + cj >= r)
            new_dig = jnp.where(hit, j, dig)
            new_r = jnp.where(hit, r - csum, r)
            new_csum = jnp.where((dig < 0) & (~hit), csum + cj, csum)
            return new_csum, new_dig, new_r
        csum, dig, r = lax.fori_loop(
            0, NBINS, body, (np.int32(0), np.int32(-1), r))
        return dig, r

    prefix = np.uint32(0)
    r = np.int32(K)
    macc = ninf
    for p in range(4):
        shift = 24 - 8 * p
        lax.fori_loop(0, NBINS, zero_hist, 0)
        if p == 0:
            macc = plsc.parallel_loop(0, NV, 1, unroll=UNROLL, carry=macc)(
                hist_pass(shift, prefix))
        else:
            plsc.parallel_loop(0, NV, 1, unroll=UNROLL, carry=ninf)(
                hist_pass(shift, prefix))
        dig, r = scan_digits(r)
        prefix = (prefix << 8) | dig.astype(jnp.uint32)

    tb = jnp.where((prefix >> 31) == np.uint32(1),
                   prefix ^ _SIGN, ~prefix)
    tvec = plsc.bitcast(jnp.broadcast_to(tb, (L,)), jnp.float32)
    mvec = jnp.broadcast_to(jnp.max(macc), (L,))

    def sum_body(i, acc):
        x = row_v[pl.ds(i * L, L)]
        e = jnp.exp(x - mvec)
        return acc + jnp.where(x >= tvec, e, 0.0)
    sacc = plsc.parallel_loop(0, NV, 1, unroll=UNROLL, carry=zvecf)(sum_body)
    svec = jnp.broadcast_to(jnp.sum(sacc), (L,))

    st_v[pl.ds(0, L)] = tvec
    st_v[pl.ds(L, L)] = mvec
    st_v[pl.ds(2 * L, L)] = svec
    pltpu.sync_copy(st_v, thr_hbm.at[pl.ds(row * 3 * L, 3 * L)])


def _body_b(logits_hbm, gumbel_hbm, thr_hbm, probs_hbm, samp_hbm,
            row_v, gbuf_v, obuf_v, st_v, samp_v):
    info = plsc.get_sparse_core_info()
    nc = info.num_cores
    wid = lax.axis_index("s") * nc + lax.axis_index("c")
    row = wid

    zero = wid * 0
    zvec = jnp.broadcast_to(zero, (L,))
    zvecf = zvec.astype(jnp.float32)
    lane = (plsc.cumsum(zvecf + 1.0) - 1.0).astype(jnp.int32)
    ninf = zvecf + _NEG_INF

    pltpu.sync_copy(logits_hbm.at[pl.ds(row * N, N)], row_v)
    pltpu.sync_copy(thr_hbm.at[pl.ds(row * 3 * L, 3 * L)], st_v)
    tvec = st_v[pl.ds(0, L)]
    mvec = st_v[pl.ds(L, L)]
    svec = st_v[pl.ds(2 * L, L)]

    bvals, bidxs = ninf, zvec
    for c in range(NCH):
        pltpu.sync_copy(gumbel_hbm.at[pl.ds(row * N + c * CHUNK, CHUNK)],
                        gbuf_v)

        def chunk_body(i, carry, c=c):
            bval, bidx = carry
            off = i * L
            x = row_v[pl.ds(c * CHUNK + off, L)]
            g = gbuf_v[pl.ds(off, L)]
            kept = x >= tvec
            pr = jnp.where(kept, jnp.exp(x - mvec) / svec, 0.0)
            obuf_v[pl.ds(off, L)] = pr
            score = jnp.where(kept, x + g, _NEG_INF)
            better = score > bval
            bval = jnp.where(better, score, bval)
            bidx = jnp.where(better, c * CHUNK + off + lane, bidx)
            return bval, bidx

        bvals, bidxs = plsc.parallel_loop(
            0, NCV, 1, unroll=FUNROLL, carry=(bvals, bidxs))(chunk_body)
        pltpu.sync_copy(obuf_v, probs_hbm.at[pl.ds(row * N + c * CHUNK, CHUNK)])

    mbest = jnp.max(bvals)
    cand = jnp.where(bvals == mbest, bidxs.astype(jnp.float32),
                     np.float32(3e7))
    samp = jnp.min(cand).astype(jnp.int32)
    samp_v[...] = jnp.broadcast_to(samp, (L,))
    pltpu.sync_copy(samp_v, samp_hbm.at[pl.ds(row * L, L)])


_sc_a = pl.kernel(
    _body_a,
    out_type=jax.ShapeDtypeStruct((B * 3 * L,), jnp.float32),
    mesh=_mesh,
    compiler_params=_params,
    scratch_types=[
        pltpu.VMEM((N,), jnp.float32),
        pltpu.VMEM((HIST,), jnp.int32),
        pltpu.VMEM((3 * L,), jnp.float32),
    ],
)

_sc_b = pl.kernel(
    _body_b,
    out_type=(
        jax.ShapeDtypeStruct((B * N,), jnp.float32),
        jax.ShapeDtypeStruct((B * L,), jnp.int32),
    ),
    mesh=_mesh,
    compiler_params=_params,
    scratch_types=[
        pltpu.VMEM((N,), jnp.float32),
        pltpu.VMEM((CHUNK,), jnp.float32),
        pltpu.VMEM((CHUNK,), jnp.float32),
        pltpu.VMEM((3 * L,), jnp.float32),
        pltpu.VMEM((L,), jnp.int32),
    ],
)


@jax.jit
def kernel(logits):
    lf = logits.reshape(-1)
    thr = _sc_a(lf)
    gumbel = jax.random.gumbel(jax.random.key(1), (B * N,), jnp.float32)
    probs, samp = _sc_b(lf, gumbel, thr)
    return probs.reshape(B, N), samp.reshape(B, L)[:, :1]


# speedup vs baseline: 27.1297x; 1.0720x over previous
import math

import jax
import jax.numpy as jnp
import numpy as np
from jax import lax
from jax.experimental import pallas as pl
from jax.experimental.pallas import tpu as pltpu
from jax.experimental.pallas import tpu_sc as plsc

B = 32
N = 100000
K = int(math.ceil((1.0 - 0.9) * N))
L = 16
NV = N // L
CHUNK = 10000
NCV = CHUNK // L
NCH = N // CHUNK
NBINS = 256
HIST = NBINS * L
UNROLL = 10
FUNROLL = 5

_SIGN = np.uint32(0x80000000)
_NEG_INF = np.float32(-np.inf)

_mesh = plsc.VectorSubcoreMesh(core_axis_name="c", subcore_axis_name="s")
_params = pltpu.CompilerParams(needs_layout_passes=False)


def _monotone_u32(x):
    ub = plsc.bitcast(x, jnp.uint32)
    return jnp.where(x < 0.0, ~ub, ub | _SIGN)


def _body_a(logits_hbm, thr_hbm, row_v, hist_v, st_v):
    info = plsc.get_sparse_core_info()
    nc = info.num_cores
    wid = lax.axis_index("s") * nc + lax.axis_index("c")
    row = wid

    zero = wid * 0
    zvec = jnp.broadcast_to(zero, (L,))
    zvecf = zvec.astype(jnp.float32)
    ones = zvec + 1
    lane = (plsc.cumsum(zvecf + 1.0) - 1.0).astype(jnp.int32)
    ninf = zvecf + _NEG_INF

    pltpu.sync_copy(logits_hbm.at[pl.ds(row * N, N)], row_v)

    def zero_hist(i, _):
        hist_v[pl.ds(i * L, L)] = zvec
        return 0

    def hist_pass(shift, prefix):
        def body(i, mx):
            if shift == 24:
                x = row_v[pl.ds(i * L, L)]
                u = _monotone_u32(x)
                row_v[pl.ds(i * L, L)] = plsc.bitcast(u, jnp.float32)
                dig = ((u >> shift) & np.uint32(0xFF)).astype(jnp.int32)
                plsc.addupdate_scatter(hist_v, [dig * L + lane], ones)
                return jnp.maximum(mx, x)
            u = plsc.bitcast(row_v[pl.ds(i * L, L)], jnp.uint32)
            dig = ((u >> shift) & np.uint32(0xFF)).astype(jnp.int32)
            ok = (u >> (shift + 8)) == prefix
            plsc.addupdate_scatter(hist_v, [dig * L + lane], ones, mask=ok)
            return mx
        return body

    def scan_digits(r):
        def body(i, carry):
            csum, dig, r = carry
            j = 255 - i
            h = hist_v[pl.ds(j * L, L)]
            cj = jnp.sum(h.astype(jnp.float32)).astype(jnp.int32)
            hit = (dig < 0) & (csum + cj >= r)
            new_dig = jnp.where(hit, j, dig)
            new_r = jnp.where(hit, r - csum, r)
            new_csum = jnp.where((dig < 0) & (~hit), csum + cj, csum)
            return new_csum, new_dig, new_r
        csum, dig, r = lax.fori_loop(
            0, NBINS, body, (np.int32(0), np.int32(-1), r))
        return dig, r

    prefix = np.uint32(0)
    r = np.int32(K)
    macc = ninf
    for p in range(4):
        shift = 24 - 8 * p
        lax.fori_loop(0, NBINS, zero_hist, 0)
        if p == 0:
            macc = plsc.parallel_loop(0, NV, 1, unroll=UNROLL, carry=macc)(
                hist_pass(shift, prefix))
        else:
            plsc.parallel_loop(0, NV, 1, unroll=UNROLL, carry=ninf)(
                hist_pass(shift, prefix))
        dig, r = scan_digits(r)
        prefix = (prefix << 8) | dig.astype(jnp.uint32)

    tb = jnp.where((prefix >> 31) == np.uint32(1),
                   prefix ^ _SIGN, ~prefix)
    tvec = plsc.bitcast(jnp.broadcast_to(tb, (L,)), jnp.float32)
    uvec = jnp.broadcast_to(prefix, (L,))
    mvec = jnp.broadcast_to(jnp.max(macc), (L,))

    def sum_body(i, acc):
        u = plsc.bitcast(row_v[pl.ds(i * L, L)], jnp.uint32)
        ub = jnp.where((u >> 31) == np.uint32(1), u ^ _SIGN, ~u)
        x = plsc.bitcast(ub, jnp.float32)
        e = jnp.exp(x - mvec)
        return acc + jnp.where(u >= uvec, e, 0.0)
    sacc = plsc.parallel_loop(0, NV, 1, unroll=UNROLL, carry=zvecf)(sum_body)
    svec = jnp.broadcast_to(jnp.sum(sacc), (L,))

    st_v[pl.ds(0, L)] = tvec
    st_v[pl.ds(L, L)] = mvec
    st_v[pl.ds(2 * L, L)] = svec
    pltpu.sync_copy(st_v, thr_hbm.at[pl.ds(row * 3 * L, 3 * L)])


def _body_b(logits_hbm, gumbel_hbm, thr_hbm, probs_hbm, samp_hbm,
            row_v, gbuf_v, obuf_v, st_v, samp_v):
    info = plsc.get_sparse_core_info()
    nc = info.num_cores
    wid = lax.axis_index("s") * nc + lax.axis_index("c")
    row = wid

    zero = wid * 0
    zvec = jnp.broadcast_to(zero, (L,))
    zvecf = zvec.astype(jnp.float32)
    lane = (plsc.cumsum(zvecf + 1.0) - 1.0).astype(jnp.int32)
    ninf = zvecf + _NEG_INF

    pltpu.sync_copy(logits_hbm.at[pl.ds(row * N, N)], row_v)
    pltpu.sync_copy(thr_hbm.at[pl.ds(row * 3 * L, 3 * L)], st_v)
    tvec = st_v[pl.ds(0, L)]
    mvec = st_v[pl.ds(L, L)]
    svec = st_v[pl.ds(2 * L, L)]

    bvals, bidxs = ninf, zvec
    for c in range(NCH):
        pltpu.sync_copy(gumbel_hbm.at[pl.ds(row * N + c * CHUNK, CHUNK)],
                        gbuf_v)

        def chunk_body(i, carry, c=c):
            bval, bidx = carry
            off = i * L
            x = row_v[pl.ds(c * CHUNK + off, L)]
            g = gbuf_v[pl.ds(off, L)]
            kept = x >= tvec
            pr = jnp.where(kept, jnp.exp(x - mvec) / svec, 0.0)
            obuf_v[pl.ds(off, L)] = pr
            score = jnp.where(kept, x + g, _NEG_INF)
            better = score > bval
            bval = jnp.where(better, score, bval)
            bidx = jnp.where(better, c * CHUNK + off + lane, bidx)
            return bval, bidx

        bvals, bidxs = plsc.parallel_loop(
            0, NCV, 1, unroll=FUNROLL, carry=(bvals, bidxs))(chunk_body)
        pltpu.sync_copy(obuf_v, probs_hbm.at[pl.ds(row * N + c * CHUNK, CHUNK)])

    mbest = jnp.max(bvals)
    cand = jnp.where(bvals == mbest, bidxs.astype(jnp.float32),
                     np.float32(3e7))
    samp = jnp.min(cand).astype(jnp.int32)
    samp_v[...] = jnp.broadcast_to(samp, (L,))
    pltpu.sync_copy(samp_v, samp_hbm.at[pl.ds(row * L, L)])


_sc_a = pl.kernel(
    _body_a,
    out_type=jax.ShapeDtypeStruct((B * 3 * L,), jnp.float32),
    mesh=_mesh,
    compiler_params=_params,
    scratch_types=[
        pltpu.VMEM((N,), jnp.float32),
        pltpu.VMEM((HIST,), jnp.int32),
        pltpu.VMEM((3 * L,), jnp.float32),
    ],
)

_sc_b = pl.kernel(
    _body_b,
    out_type=(
        jax.ShapeDtypeStruct((B * N,), jnp.float32),
        jax.ShapeDtypeStruct((B * L,), jnp.int32),
    ),
    mesh=_mesh,
    compiler_params=_params,
    scratch_types=[
        pltpu.VMEM((N,), jnp.float32),
        pltpu.VMEM((CHUNK,), jnp.float32),
        pltpu.VMEM((CHUNK,), jnp.float32),
        pltpu.VMEM((3 * L,), jnp.float32),
        pltpu.VMEM((L,), jnp.int32),
    ],
)


@jax.jit
def kernel(logits):
    lf = logits.reshape(-1)
    thr = _sc_a(lf)
    gumbel = jax.random.gumbel(jax.random.key(1), (B * N,), jnp.float32)
    probs, samp = _sc_b(lf, gumbel, thr)
    return probs.reshape(B, N), samp.reshape(B, L)[:, :1]


# speedup vs baseline: 31.2691x; 1.1526x over previous
import math

import jax
import jax.numpy as jnp
import numpy as np
from jax import lax
from jax.experimental import pallas as pl
from jax.experimental.pallas import tpu as pltpu
from jax.experimental.pallas import tpu_sc as plsc

B = 32
N = 100000
K = int(math.ceil((1.0 - 0.9) * N))
L = 16
NV = N // L
CHUNK = 10000
NCV = CHUNK // L
NCH = N // CHUNK
NBINS = 256
HIST = NBINS * L
UNROLL = 10
FUNROLL = 5

_SIGN = np.uint32(0x80000000)
_NEG_INF = np.float32(-np.inf)

_mesh = plsc.VectorSubcoreMesh(core_axis_name="c", subcore_axis_name="s")
_params = pltpu.CompilerParams(needs_layout_passes=False)


def _monotone_u32(x):
    ub = plsc.bitcast(x, jnp.uint32)
    return jnp.where(x < 0.0, ~ub, ub | _SIGN)


def _body_a(logits_hbm, thr_hbm, row_v, hist_v, st_v, rsem0, rsem1):
    info = plsc.get_sparse_core_info()
    nc = info.num_cores
    wid = lax.axis_index("s") * nc + lax.axis_index("c")
    row = wid

    zero = wid * 0
    zvec = jnp.broadcast_to(zero, (L,))
    zvecf = zvec.astype(jnp.float32)
    ones = zvec + 1
    lane = (plsc.cumsum(zvecf + 1.0) - 1.0).astype(jnp.int32)
    ninf = zvecf + _NEG_INF

    def rcopy(c, sem):
        return pltpu.make_async_copy(
            logits_hbm.at[pl.ds(row * N + c * CHUNK, CHUNK)],
            row_v.at[pl.ds(c * CHUNK, CHUNK)], sem)

    rcopy(0, rsem0).start()
    rcopy(1, rsem1).start()

    def zero_hist(i, _):
        hist_v[pl.ds(i * L, L)] = zvec
        return 0

    def hist_pass(shift, prefix):
        def body(i, mx):
            if shift == 24:
                x = row_v[pl.ds(i * L, L)]
                u = _monotone_u32(x)
                row_v[pl.ds(i * L, L)] = plsc.bitcast(u, jnp.float32)
                dig = ((u >> shift) & np.uint32(0xFF)).astype(jnp.int32)
                plsc.addupdate_scatter(hist_v, [dig * L + lane], ones)
                return jnp.maximum(mx, x)
            u = plsc.bitcast(row_v[pl.ds(i * L, L)], jnp.uint32)
            dig = ((u >> shift) & np.uint32(0xFF)).astype(jnp.int32)
            ok = (u >> (shift + 8)) == prefix
            plsc.addupdate_scatter(hist_v, [dig * L + lane], ones, mask=ok)
            return mx
        return body

    def scan_digits(r):
        def body(i, carry):
            csum, dig, r = carry
            j = 255 - i
            h = hist_v[pl.ds(j * L, L)]
            cj = jnp.sum(h.astype(jnp.float32)).astype(jnp.int32)
            hit = (dig < 0) & (csum + cj >= r)
            new_dig = jnp.where(hit, j, dig)
            new_r = jnp.where(hit, r - csum, r)
            new_csum = jnp.where((dig < 0) & (~hit), csum + cj, csum)
            return new_csum, new_dig, new_r
        csum, dig, r = lax.fori_loop(
            0, NBINS, body, (np.int32(0), np.int32(-1), r))
        return dig, r

    prefix = np.uint32(0)
    r = np.int32(K)
    macc = ninf
    for p in range(4):
        shift = 24 - 8 * p
        lax.fori_loop(0, NBINS, zero_hist, 0)
        if p == 0:
            for c in range(NCH):
                rcopy(c, (rsem0, rsem1)[c % 2]).wait()
                if c + 2 < NCH:
                    rcopy(c + 2, (rsem0, rsem1)[c % 2]).start()
                macc = plsc.parallel_loop(
                    c * NCV, (c + 1) * NCV, 1, unroll=UNROLL, carry=macc)(
                    hist_pass(shift, prefix))
        else:
            plsc.parallel_loop(0, NV, 1, unroll=UNROLL, carry=ninf)(
                hist_pass(shift, prefix))
        dig, r = scan_digits(r)
        prefix = (prefix << 8) | dig.astype(jnp.uint32)

    tb = jnp.where((prefix >> 31) == np.uint32(1),
                   prefix ^ _SIGN, ~prefix)
    tvec = plsc.bitcast(jnp.broadcast_to(tb, (L,)), jnp.float32)
    uvec = jnp.broadcast_to(prefix, (L,))
    mvec = jnp.broadcast_to(jnp.max(macc), (L,))

    def sum_body(i, acc):
        u = plsc.bitcast(row_v[pl.ds(i * L, L)], jnp.uint32)
        ub = jnp.where((u >> 31) == np.uint32(1), u ^ _SIGN, ~u)
        x = plsc.bitcast(ub, jnp.float32)
        e = jnp.exp(x - mvec)
        return acc + jnp.where(u >= uvec, e, 0.0)
    sacc = plsc.parallel_loop(0, NV, 1, unroll=UNROLL, carry=zvecf)(sum_body)
    svec = jnp.broadcast_to(jnp.sum(sacc), (L,))

    st_v[pl.ds(0, L)] = tvec
    st_v[pl.ds(L, L)] = mvec
    st_v[pl.ds(2 * L, L)] = svec
    pltpu.sync_copy(st_v, thr_hbm.at[pl.ds(row * 3 * L, 3 * L)])


def _body_b(logits_hbm, gumbel_hbm, thr_hbm, probs_hbm, samp_hbm,
            rbufs, gbufs, obufs, st_v, samp_v, rsems, gsems, osems):
    info = plsc.get_sparse_core_info()
    nc = info.num_cores
    wid = lax.axis_index("s") * nc + lax.axis_index("c")
    row = wid

    zero = wid * 0
    zvec = jnp.broadcast_to(zero, (L,))
    zvecf = zvec.astype(jnp.float32)
    lane = (plsc.cumsum(zvecf + 1.0) - 1.0).astype(jnp.int32)
    ninf = zvecf + _NEG_INF

    def rcopy(c):
        return pltpu.make_async_copy(
            logits_hbm.at[pl.ds(row * N + c * CHUNK, CHUNK)],
            rbufs[c % 2], rsems[c % 2])

    def gcopy(c):
        return pltpu.make_async_copy(
            gumbel_hbm.at[pl.ds(row * N + c * CHUNK, CHUNK)],
            gbufs[c % 2], gsems[c % 2])

    def ocopy(c):
        return pltpu.make_async_copy(
            obufs[c % 2],
            probs_hbm.at[pl.ds(row * N + c * CHUNK, CHUNK)], osems[c % 2])

    rcopy(0).start()
    gcopy(0).start()
    pltpu.sync_copy(thr_hbm.at[pl.ds(row * 3 * L, 3 * L)], st_v)
    tvec = st_v[pl.ds(0, L)]
    mvec = st_v[pl.ds(L, L)]
    svec = st_v[pl.ds(2 * L, L)]

    bvals, bidxs = ninf, zvec
    for c in range(NCH):
        if c + 1 < NCH:
            rcopy(c + 1).start()
            gcopy(c + 1).start()
        rcopy(c).wait()
        gcopy(c).wait()
        if c >= 2:
            ocopy(c - 2).wait()
        rbuf, gbuf, obuf = rbufs[c % 2], gbufs[c % 2], obufs[c % 2]

        def chunk_body(i, carry, c=c, rbuf=rbuf, gbuf=gbuf, obuf=obuf):
            bval, bidx = carry
            off = i * L
            x = rbuf[pl.ds(off, L)]
            g = gbuf[pl.ds(off, L)]
            kept = x >= tvec
            pr = jnp.where(kept, jnp.exp(x - mvec) / svec, 0.0)
            obuf[pl.ds(off, L)] = pr
            score = jnp.where(kept, x + g, _NEG_INF)
            better = score > bval
            bval = jnp.where(better, score, bval)
            bidx = jnp.where(better, c * CHUNK + off + lane, bidx)
            return bval, bidx

        bvals, bidxs = plsc.parallel_loop(
            0, NCV, 1, unroll=FUNROLL, carry=(bvals, bidxs))(chunk_body)
        ocopy(c).start()
    ocopy(NCH - 2).wait()
    ocopy(NCH - 1).wait()

    mbest = jnp.max(bvals)
    cand = jnp.where(bvals == mbest, bidxs.astype(jnp.float32),
                     np.float32(3e7))
    samp = jnp.min(cand).astype(jnp.int32)
    samp_v[...] = jnp.broadcast_to(samp, (L,))
    pltpu.sync_copy(samp_v, samp_hbm.at[pl.ds(row * L, L)])


_sc_a = pl.kernel(
    _body_a,
    out_type=jax.ShapeDtypeStruct((B * 3 * L,), jnp.float32),
    mesh=_mesh,
    compiler_params=_params,
    scratch_types=[
        pltpu.VMEM((N,), jnp.float32),
        pltpu.VMEM((HIST,), jnp.int32),
        pltpu.VMEM((3 * L,), jnp.float32),
        pltpu.SemaphoreType.DMA,
        pltpu.SemaphoreType.DMA,
    ],
)

_sc_b = pl.kernel(
    _body_b,
    out_type=(
        jax.ShapeDtypeStruct((B * N,), jnp.float32),
        jax.ShapeDtypeStruct((B * L,), jnp.int32),
    ),
    mesh=_mesh,
    compiler_params=_params,
    scratch_types=[
        [pltpu.VMEM((CHUNK,), jnp.float32)] * 2,
        [pltpu.VMEM((CHUNK,), jnp.float32)] * 2,
        [pltpu.VMEM((CHUNK,), jnp.float32)] * 2,
        pltpu.VMEM((3 * L,), jnp.float32),
        pltpu.VMEM((L,), jnp.int32),
        [pltpu.SemaphoreType.DMA] * 2,
        [pltpu.SemaphoreType.DMA] * 2,
        [pltpu.SemaphoreType.DMA] * 2,
    ],
)


@jax.jit
def kernel(logits):
    lf = logits.reshape(-1)
    thr = _sc_a(lf)
    gumbel = jax.random.gumbel(jax.random.key(1), (B * N,), jnp.float32)
    probs, samp = _sc_b(lf, gumbel, thr)
    return probs.reshape(B, N), samp.reshape(B, L)[:, :1]


# speedup vs baseline: 32.3850x; 1.0357x over previous
import math

import jax
import jax.numpy as jnp
import numpy as np
from jax import lax
from jax.experimental import pallas as pl
from jax.experimental.pallas import tpu as pltpu
from jax.experimental.pallas import tpu_sc as plsc

B = 32
N = 100000
K = int(math.ceil((1.0 - 0.9) * N))
L = 16
NV = N // L
CHUNK = 10000
NCV = CHUNK // L
NCH = N // CHUNK
NBINS = 256
HIST = NBINS * L
UNROLL = 10
FUNROLL = 5

_SIGN = np.uint32(0x80000000)
_NEG_INF = np.float32(-np.inf)

_mesh = plsc.VectorSubcoreMesh(core_axis_name="c", subcore_axis_name="s")
_params = pltpu.CompilerParams(needs_layout_passes=False)


def _monotone_u32(x):
    ub = plsc.bitcast(x, jnp.uint32)
    return jnp.where(x < 0.0, ~ub, ub | _SIGN)


def _body_a(logits_hbm, thr_hbm, row_v, hist_v, st_v, rsem0, rsem1):
    info = plsc.get_sparse_core_info()
    nc = info.num_cores
    wid = lax.axis_index("s") * nc + lax.axis_index("c")
    row = wid

    zero = wid * 0
    zvec = jnp.broadcast_to(zero, (L,))
    zvecf = zvec.astype(jnp.float32)
    ones = zvec + 1
    lane = (plsc.cumsum(zvecf + 1.0) - 1.0).astype(jnp.int32)
    ninf = zvecf + _NEG_INF

    def rcopy(c, sem):
        return pltpu.make_async_copy(
            logits_hbm.at[pl.ds(row * N + c * CHUNK, CHUNK)],
            row_v.at[pl.ds(c * CHUNK, CHUNK)], sem)

    rcopy(0, rsem0).start()
    rcopy(1, rsem1).start()

    def zero_hist(i, _):
        hist_v[pl.ds(i * L, L)] = zvec
        return 0

    def hist_pass(shift, prefix):
        def body(i, mx):
            if shift == 24:
                x = row_v[pl.ds(i * L, L)]
                u = _monotone_u32(x)
                row_v[pl.ds(i * L, L)] = plsc.bitcast(u, jnp.float32)
                dig = ((u >> shift) & np.uint32(0xFF)).astype(jnp.int32)
                plsc.addupdate_scatter(hist_v, [dig * L + lane], ones)
                return jnp.maximum(mx, x)
            u = plsc.bitcast(row_v[pl.ds(i * L, L)], jnp.uint32)
            dig = ((u >> shift) & np.uint32(0xFF)).astype(jnp.int32)
            ok = (u >> (shift + 8)) == prefix
            plsc.addupdate_scatter(hist_v, [dig * L + lane], ones, mask=ok)
            return mx
        return body

    def scan_digits(r):
        def body(i, carry):
            csum, dig, r = carry
            j = 255 - i
            h = hist_v[pl.ds(j * L, L)]
            hist_v[pl.ds(j * L, L)] = zvec
            cj = jnp.sum(h.astype(jnp.float32)).astype(jnp.int32)
            hit = (dig < 0) & (csum + cj >= r)
            new_dig = jnp.where(hit, j, dig)
            new_r = jnp.where(hit, r - csum, r)
            new_csum = jnp.where((dig < 0) & (~hit), csum + cj, csum)
            return new_csum, new_dig, new_r
        csum, dig, r = plsc.parallel_loop(
            0, NBINS, 1, unroll=8,
            carry=(zero, zero - 1, r))(body)
        return dig, r

    prefix = np.uint32(0)
    r = zero + K
    macc = ninf
    for p in range(4):
        shift = 24 - 8 * p
        if p == 0:
            lax.fori_loop(0, NBINS, zero_hist, 0)
        if p == 0:
            for c in range(NCH):
                rcopy(c, (rsem0, rsem1)[c % 2]).wait()
                if c + 2 < NCH:
                    rcopy(c + 2, (rsem0, rsem1)[c % 2]).start()
                macc = plsc.parallel_loop(
                    c * NCV, (c + 1) * NCV, 1, unroll=UNROLL, carry=macc)(
                    hist_pass(shift, prefix))
        else:
            plsc.parallel_loop(0, NV, 1, unroll=UNROLL, carry=ninf)(
                hist_pass(shift, prefix))
        dig, r = scan_digits(r)
        prefix = (prefix << 8) | dig.astype(jnp.uint32)

    tb = jnp.where((prefix >> 31) == np.uint32(1),
                   prefix ^ _SIGN, ~prefix)
    tvec = plsc.bitcast(jnp.broadcast_to(tb, (L,)), jnp.float32)
    uvec = jnp.broadcast_to(prefix, (L,))
    mvec = jnp.broadcast_to(jnp.max(macc), (L,))

    def sum_body(i, acc):
        u = plsc.bitcast(row_v[pl.ds(i * L, L)], jnp.uint32)
        ub = jnp.where((u >> 31) == np.uint32(1), u ^ _SIGN, ~u)
        x = plsc.bitcast(ub, jnp.float32)
        e = jnp.exp(x - mvec)
        return acc + jnp.where(u >= uvec, e, 0.0)
    sacc = plsc.parallel_loop(0, NV, 1, unroll=UNROLL, carry=zvecf)(sum_body)
    svec = jnp.broadcast_to(jnp.sum(sacc), (L,))

    st_v[pl.ds(0, L)] = tvec
    st_v[pl.ds(L, L)] = mvec
    st_v[pl.ds(2 * L, L)] = svec
    pltpu.sync_copy(st_v, thr_hbm.at[pl.ds(row * 3 * L, 3 * L)])


def _body_b(logits_hbm, gumbel_hbm, thr_hbm, probs_hbm, samp_hbm,
            rbufs, gbufs, obufs, st_v, samp_v, rsems, gsems, osems):
    info = plsc.get_sparse_core_info()
    nc = info.num_cores
    wid = lax.axis_index("s") * nc + lax.axis_index("c")
    row = wid

    zero = wid * 0
    zvec = jnp.broadcast_to(zero, (L,))
    zvecf = zvec.astype(jnp.float32)
    lane = (plsc.cumsum(zvecf + 1.0) - 1.0).astype(jnp.int32)
    ninf = zvecf + _NEG_INF

    def rcopy(c):
        return pltpu.make_async_copy(
            logits_hbm.at[pl.ds(row * N + c * CHUNK, CHUNK)],
            rbufs[c % 2], rsems[c % 2])

    def gcopy(c):
        return pltpu.make_async_copy(
            gumbel_hbm.at[pl.ds(row * N + c * CHUNK, CHUNK)],
            gbufs[c % 2], gsems[c % 2])

    def ocopy(c):
        return pltpu.make_async_copy(
            obufs[c % 2],
            probs_hbm.at[pl.ds(row * N + c * CHUNK, CHUNK)], osems[c % 2])

    rcopy(0).start()
    gcopy(0).start()
    pltpu.sync_copy(thr_hbm.at[pl.ds(row * 3 * L, 3 * L)], st_v)
    tvec = st_v[pl.ds(0, L)]
    mvec = st_v[pl.ds(L, L)]
    svec = st_v[pl.ds(2 * L, L)]

    bvals, bidxs = ninf, zvec
    for c in range(NCH):
        if c + 1 < NCH:
            rcopy(c + 1).start()
            gcopy(c + 1).start()
        rcopy(c).wait()
        gcopy(c).wait()
        if c >= 2:
            ocopy(c - 2).wait()
        rbuf, gbuf, obuf = rbufs[c % 2], gbufs[c % 2], obufs[c % 2]

        def chunk_body(i, carry, c=c, rbuf=rbuf, gbuf=gbuf, obuf=obuf):
            bval, bidx = carry
            off = i * L
            x = rbuf[pl.ds(off, L)]
            g = gbuf[pl.ds(off, L)]
            kept = x >= tvec
            pr = jnp.where(kept, jnp.exp(x - mvec) / svec, 0.0)
            obuf[pl.ds(off, L)] = pr
            score = jnp.where(kept, x + g, _NEG_INF)
            better = score > bval
            bval = jnp.where(better, score, bval)
            bidx = jnp.where(better, c * CHUNK + off + lane, bidx)
            return bval, bidx

        bvals, bidxs = plsc.parallel_loop(
            0, NCV, 1, unroll=FUNROLL, carry=(bvals, bidxs))(chunk_body)
        ocopy(c).start()
    ocopy(NCH - 2).wait()
    ocopy(NCH - 1).wait()

    mbest = jnp.max(bvals)
    cand = jnp.where(bvals == mbest, bidxs.astype(jnp.float32),
                     np.float32(3e7))
    samp = jnp.min(cand).astype(jnp.int32)
    samp_v[...] = jnp.broadcast_to(samp, (L,))
    pltpu.sync_copy(samp_v, samp_hbm.at[pl.ds(row * L, L)])


_sc_a = pl.kernel(
    _body_a,
    out_type=jax.ShapeDtypeStruct((B * 3 * L,), jnp.float32),
    mesh=_mesh,
    compiler_params=_params,
    scratch_types=[
        pltpu.VMEM((N,), jnp.float32),
        pltpu.VMEM((HIST,), jnp.int32),
        pltpu.VMEM((3 * L,), jnp.float32),
        pltpu.SemaphoreType.DMA,
        pltpu.SemaphoreType.DMA,
    ],
)

_sc_b = pl.kernel(
    _body_b,
    out_type=(
        jax.ShapeDtypeStruct((B * N,), jnp.float32),
        jax.ShapeDtypeStruct((B * L,), jnp.int32),
    ),
    mesh=_mesh,
    compiler_params=_params,
    scratch_types=[
        [pltpu.VMEM((CHUNK,), jnp.float32)] * 2,
        [pltpu.VMEM((CHUNK,), jnp.float32)] * 2,
        [pltpu.VMEM((CHUNK,), jnp.float32)] * 2,
        pltpu.VMEM((3 * L,), jnp.float32),
        pltpu.VMEM((L,), jnp.int32),
        [pltpu.SemaphoreType.DMA] * 2,
        [pltpu.SemaphoreType.DMA] * 2,
        [pltpu.SemaphoreType.DMA] * 2,
    ],
)


@jax.jit
def kernel(logits):
    lf = logits.reshape(-1)
    thr = _sc_a(lf)
    gumbel = jax.random.gumbel(jax.random.key(1), (B * N,), jnp.float32)
    probs, samp = _sc_b(lf, gumbel, thr)
    return probs.reshape(B, N), samp.reshape(B, L)[:, :1]
